# baseline (speedup 1.0000x reference)
import jax
import jax.numpy as jnp
from jax.experimental import pallas as pl
from jax.experimental.pallas import tpu as pltpu

_B, _C, _H, _W = 128, 5, 128, 128
_BB = 8
_G = _B // _BB


def _loss_kernel(a_ref, b_ref, pab_ref, pba_ref, o_ref, acc_ref):
    i = pl.program_id(0)
    acc2d = jnp.zeros((_H, _W), jnp.float32)
    for b in range(_BB):
        cell = None
        for c in range(_C):
            d1 = a_ref[b, c] - pab_ref[b, c]
            d2 = b_ref[b, c] - pba_ref[b, c]
            t = d1 * d1 + d2 * d2
            cell = t if cell is None else cell + t
        acc2d = acc2d + jnp.where(a_ref[b, 0] != 0, cell, 0.0)
    part = jnp.sum(acc2d, axis=0, keepdims=True)

    @pl.when(i == 0)
    def _():
        acc_ref[...] = part

    @pl.when(i > 0)
    def _():
        acc_ref[...] += part

    @pl.when(i == _G - 1)
    def _():
        o_ref[0, 0] = jnp.sum(acc_ref[...])


def kernel(labela, labelb, pred_ab, pred_ba):
    in_spec = pl.BlockSpec((_BB, _C, _H, _W), lambda i: (i, 0, 0, 0))
    out = pl.pallas_call(
        _loss_kernel,
        out_shape=jax.ShapeDtypeStruct((1, 1), jnp.float32),
        grid=(_G,),
        in_specs=[in_spec, in_spec, in_spec, in_spec],
        out_specs=pl.BlockSpec(memory_space=pltpu.SMEM),
        scratch_shapes=[pltpu.VMEM((1, _W), jnp.float32)],
        compiler_params=pltpu.CompilerParams(
            dimension_semantics=("arbitrary",),
        ),
        name="yolo_masked_sse",
    )(labela, labelb, pred_ab, pred_ba)
    return out[0, 0]
